# baseline (speedup 1.0000x reference)
import jax
import jax.numpy as jnp
from jax import lax
from jax.experimental import pallas as pl
from jax.experimental.pallas import tpu as pltpu

_V = 8192
_D = 1024
_B = 4 * 8192
_CHUNK = 512


def _tc_body(idx_ref, table_ref, out_ref):
    base = pl.program_id(0) * _CHUNK

    def inner(j, carry):
        r = idx_ref[base + j]
        out_ref[pl.ds(j, 1)] = table_ref[pl.ds(r, 1)]
        return carry

    lax.fori_loop(0, _CHUNK, inner, 0, unroll=8)


def kernel(position_ids, table):
    idx = position_ids.reshape(_B).astype(jnp.int32)
    t3 = table.reshape(_V, 8, 128)
    spec = pltpu.PrefetchScalarGridSpec(
        num_scalar_prefetch=1,
        grid=(_B // _CHUNK,),
        in_specs=[pl.BlockSpec((_V, 8, 128), lambda i, idx_ref: (0, 0, 0))],
        out_specs=pl.BlockSpec(
            (_CHUNK, 8, 128), lambda i, idx_ref: (i, 0, 0)),
    )
    out = pl.pallas_call(
        _tc_body,
        grid_spec=spec,
        out_shape=jax.ShapeDtypeStruct((_B, 8, 128), jnp.float32),
        compiler_params=pltpu.CompilerParams(
            dimension_semantics=("arbitrary",)),
    )(idx, t3)
    return out.reshape(position_ids.shape + (_D,))


# speedup vs baseline: 2.2912x; 2.2912x over previous
import functools

import jax
import jax.numpy as jnp
from jax import lax
from jax.experimental import pallas as pl
from jax.experimental.pallas import tpu as pltpu
from jax.experimental.pallas import tpu_sc as plsc

_V = 8192
_D = 1024
_B = 4 * 8192
_NW = 32
_BPW = _B // _NW
_C = 8
_NBUF = 8
_NCHUNK = _BPW // _C
_NGROUP = _NCHUNK // _NBUF

_mesh = plsc.VectorSubcoreMesh(core_axis_name="c", subcore_axis_name="s")


@functools.partial(
    pl.kernel,
    mesh=_mesh,
    out_type=jax.ShapeDtypeStruct((_B, _D), jnp.float32),
    scratch_types=[
        pltpu.VMEM((_NCHUNK, _C), jnp.int32),
        *[pltpu.VMEM((_C, _D), jnp.float32) for _ in range(_NBUF)],
        *[pltpu.SemaphoreType.DMA for _ in range(2 * _NBUF)],
    ],
)
def _gather_rows(idx_hbm, table_hbm, out_hbm, idx_v, *bufs_and_sems):
    bufs = bufs_and_sems[:_NBUF]
    sem_g = bufs_and_sems[_NBUF:2 * _NBUF]
    sem_s = bufs_and_sems[2 * _NBUF:]

    cid = lax.axis_index("c")
    sid = lax.axis_index("s")
    wid = sid * 2 + cid
    base = wid * _BPW
    pltpu.sync_copy(idx_hbm.at[wid], idx_v)

    def out_at(c):
        return out_hbm.at[pl.ds(base + c * _C, _C)]

    for b in range(_NBUF):
        pltpu.async_copy(table_hbm.at[idx_v.at[b]], bufs[b], sem_g[b])

    def body(g, carry):
        c0 = g * _NBUF
        for b in range(_NBUF):
            c = c0 + b
            pltpu.make_async_copy(
                table_hbm.at[idx_v.at[c]], bufs[b], sem_g[b]).wait()
            pltpu.async_copy(bufs[b], out_at(c), sem_s[b])
        for b in range(_NBUF):
            c = c0 + b
            pltpu.make_async_copy(bufs[b], out_at(c), sem_s[b]).wait()
            pltpu.async_copy(
                table_hbm.at[idx_v.at[c + _NBUF]], bufs[b], sem_g[b])
        return carry

    lax.fori_loop(0, _NGROUP - 1, body, 0)

    c0 = (_NGROUP - 1) * _NBUF
    for b in range(_NBUF):
        c = c0 + b
        pltpu.make_async_copy(
            table_hbm.at[idx_v.at[c]], bufs[b], sem_g[b]).wait()
        pltpu.async_copy(bufs[b], out_at(c), sem_s[b])
    for b in range(_NBUF):
        c = c0 + b
        pltpu.make_async_copy(bufs[b], out_at(c), sem_s[b]).wait()


def kernel(position_ids, table):
    idx = position_ids.reshape(_NW, _NCHUNK, _C).astype(jnp.int32)
    out = _gather_rows(idx, table)
    return out.reshape(position_ids.shape + (_D,))
